# baseline (speedup 1.0000x reference)
import functools

import jax
import jax.numpy as jnp
from jax import lax
from jax.experimental import pallas as pl
from jax.experimental.pallas import tpu as pltpu
from jax.experimental.pallas import tpu_sc as plsc

_NC, _NS = 2, 16
_NW = _NC * _NS


@functools.lru_cache(maxsize=None)
def _make_gather(total, dim):
    n = total // _NW
    assert n % 16 == 0
    mesh = plsc.VectorSubcoreMesh(core_axis_name="c", subcore_axis_name="s")

    @functools.partial(
        pl.kernel,
        out_type=jax.ShapeDtypeStruct((total, dim), jnp.float32),
        mesh=mesh,
        scratch_types=[
            pltpu.VMEM((n,), jnp.int32),
            pltpu.SemaphoreType.DMA,
        ],
    )
    def gather_k(ids_hbm, table_hbm, out_hbm, ids_v, sem):
        wid = lax.axis_index("s") * _NC + lax.axis_index("c")
        base = wid * n
        pltpu.sync_copy(ids_hbm.at[pl.ds(base, n)], ids_v)

        def grp_body(g, carry):
            off = pl.multiple_of(g * 16, 16)
            vec = ids_v[pl.ds(off, 16)]
            for u in range(16):
                row = vec[u]
                pltpu.async_copy(
                    table_hbm.at[pl.ds(row, 1)],
                    out_hbm.at[pl.ds(base + off + u, 1)],
                    sem,
                )
            for u in range(16):
                pltpu.make_async_copy(
                    table_hbm.at[pl.ds(0, 1)],
                    out_hbm.at[pl.ds(0, 1)],
                    sem,
                ).wait()
            return carry

        lax.fori_loop(0, n // 16, grp_body, 0)

    return gather_k


def kernel(input, table):
    ids = input
    b, l = ids.shape
    total = b * l
    dim = table.shape[1]
    flat = ids.reshape(total)
    out = _make_gather(total, dim)(flat, table)
    return out.reshape(b, l, dim)


# speedup vs baseline: 4.7012x; 4.7012x over previous
import functools

import jax
import jax.numpy as jnp
from jax import lax
from jax.experimental import pallas as pl
from jax.experimental.pallas import tpu as pltpu
from jax.experimental.pallas import tpu_sc as plsc

_NC, _NS = 2, 16
_NW = _NC * _NS


@functools.lru_cache(maxsize=None)
def _make_gather(total, dim, n_chunks):
    b_per_w = total // _NW
    chunk = b_per_w // n_chunks
    mesh = plsc.VectorSubcoreMesh(core_axis_name="c", subcore_axis_name="s")

    @functools.partial(
        pl.kernel,
        out_type=jax.ShapeDtypeStruct((total, dim), jnp.float32),
        mesh=mesh,
        compiler_params=pltpu.CompilerParams(use_tc_tiling_on_sc=False),
        scratch_types=[
            pltpu.VMEM((b_per_w,), jnp.int32),
            pltpu.VMEM((2, chunk, dim), jnp.float32),
            pltpu.SemaphoreType.DMA,
            pltpu.SemaphoreType.DMA,
            pltpu.SemaphoreType.DMA,
            pltpu.SemaphoreType.DMA,
        ],
    )
    def gather_k(ids_hbm, table_hbm, out_hbm, idx_v, rows_v, g0, g1, o0, o1):
        wid = lax.axis_index("s") * _NC + lax.axis_index("c")
        base = wid * b_per_w
        pltpu.sync_copy(ids_hbm.at[pl.ds(base, b_per_w)], idx_v)
        gsems, osems = (g0, g1), (o0, o1)
        gcp, ocp = [None, None], [None, None]

        def gather(j):
            return pltpu.async_copy(
                table_hbm.at[idx_v.at[pl.ds(j * chunk, chunk)]],
                rows_v.at[j % 2],
                gsems[j % 2],
            )

        gcp[0] = gather(0)
        for j in range(n_chunks):
            b = j % 2
            if j + 1 < n_chunks:
                nb = (j + 1) % 2
                if ocp[nb] is not None:
                    ocp[nb].wait()
                gcp[nb] = gather(j + 1)
            gcp[b].wait()
            ocp[b] = pltpu.async_copy(
                rows_v.at[b],
                out_hbm.at[pl.ds(base + j * chunk, chunk)],
                osems[b],
            )
        ocp[(n_chunks - 1) % 2].wait()
        if n_chunks > 1:
            ocp[(n_chunks - 2) % 2].wait()

    return gather_k


def kernel(input, table):
    ids = input
    b, l = ids.shape
    total = b * l
    dim = table.shape[1]
    flat = ids.reshape(total)
    out = _make_gather(total, dim, 8)(flat, table)
    return out.reshape(b, l, dim)
